# baseline (speedup 1.0000x reference)
import functools

import jax
import jax.numpy as jnp
from jax import lax
from jax.experimental import pallas as pl
from jax.experimental.pallas import tpu as pltpu
from jax.experimental.pallas import tpu_sc as plsc

R, T = 4096, 200
D = 32
NC, NS = 2, 16
NW = NC * NS
IPW = R // NW
NBUF = 8
NGROUPS = T // NBUF
L = 16


_mesh = plsc.VectorSubcoreMesh(core_axis_name="c", subcore_axis_name="s")


@functools.partial(
    pl.kernel,
    out_type=jax.ShapeDtypeStruct((T, D, R), jnp.float32),
    mesh=_mesh,
    compiler_params=pltpu.CompilerParams(use_tc_tiling_on_sc=False, needs_layout_passes=False),
    scratch_types=[
        pltpu.VMEM((T, IPW), jnp.int32),
        pltpu.VMEM((NBUF, IPW, D), jnp.float32),
        pltpu.VMEM((NBUF, D, IPW), jnp.float32),
        pltpu.SemaphoreType.DMA((NBUF,)),
        pltpu.SemaphoreType.DMA((NBUF,)),
    ],
)
def _embed_sc(idx_hbm, table_hbm, out_hbm, idx_v, rows_v, trans_v, gsem, ssem):
    wid = lax.axis_index("s") * NC + lax.axis_index("c")
    i0 = wid * IPW
    pltpu.sync_copy(idx_hbm.at[:, pl.ds(i0, IPW)], idx_v)

    def gather(j, b):
        pltpu.async_copy(table_hbm.at[idx_v.at[j]], rows_v.at[b], gsem.at[b])

    def transpose(b):
        rows = rows_v.at[b]
        trans = trans_v.at[b]

        def per_dim(d, carry):
            for m in range(IPW // L):
                i_idx = lax.iota(jnp.int32, L) + m * L
                d_idx = jnp.full((L,), d, jnp.int32)
                trans[d, pl.ds(m * L, L)] = plsc.load_gather(rows, [i_idx, d_idx])
            return carry

        lax.fori_loop(0, D, per_dim, 0)

    for b in range(NBUF):
        gather(b, b)

    def group(g, carry):
        base = g * NBUF
        for b in range(NBUF):
            pltpu.make_async_copy(
                table_hbm.at[idx_v.at[base + b]], rows_v.at[b], gsem.at[b]
            ).wait()
            transpose(b)
            pltpu.async_copy(
                trans_v.at[b], out_hbm.at[base + b, :, pl.ds(i0, IPW)], ssem.at[b]
            )
        for b in range(NBUF):
            pltpu.make_async_copy(
                trans_v.at[b], out_hbm.at[base + b, :, pl.ds(i0, IPW)], ssem.at[b]
            ).wait()

            @pl.when(g + 1 < NGROUPS)
            def _():
                gather(base + NBUF + b, b)

        return carry

    lax.fori_loop(0, NGROUPS, group, 0)


def kernel(token_ids, weight):
    out_p = _embed_sc(token_ids.T, weight)
    return jnp.transpose(out_p, (2, 0, 1))
